# baseline (speedup 1.0000x reference)
import jax
import jax.numpy as jnp
from jax import lax
from jax.experimental import pallas as pl
import jax.experimental.pallas.tpu as pltpu
from jax.experimental.pallas import tpu_sc as plsc

_WAY = 5
_CPW = 24
_SB = 2


def _sc_body(x_hbm, s_hbm, o_hbm, xb, sb, yb, s_sem, in_sem, out_sem):
    N, c, hw = x_hbm.shape
    G = N // _WAY
    wid = lax.axis_index("s") * 2 + lax.axis_index("c")
    c0 = wid * _CPW
    nvr = (hw + 15) // 16

    def start_in(k, b):
        pltpu.make_async_copy(x_hbm.at[pl.ds(k * _SB, _SB), pl.ds(c0, _CPW)],
                              xb.at[b], in_sem.at[b]).start()

    def wait_in(k, b):
        pltpu.make_async_copy(x_hbm.at[pl.ds(k * _SB, _SB), pl.ds(c0, _CPW)],
                              xb.at[b], in_sem.at[b]).wait()

    def out_copy(k, b, j):
        n0 = k * _SB
        way = n0 // G
        g0 = lax.rem(n0, G)
        return pltpu.make_async_copy(
            yb.at[b],
            o_hbm.at[way, j, pl.ds(g0, _SB), pl.ds(c0, _CPW)],
            out_sem.at[b])

    pltpu.make_async_copy(s_hbm, sb, s_sem).start()
    start_in(0, 0)
    start_in(1, 1)
    pltpu.make_async_copy(s_hbm, sb, s_sem).wait()

    lanes = [jnp.minimum(lax.iota(jnp.int32, 16) + v * 16,
                         hw - 16 + lax.iota(jnp.int32, 16))
             for v in range(nvr)]

    K = N // _SB
    @pl.loop(0, K // 2)
    def _(g2):
        for b in range(2):
            k = g2 * 2 + b
            wait_in(k, b)

            @pl.when(k >= 2)
            def _():
                for j in range(2):
                    out_copy(k - 2, b, j).wait()

            for i in range(_SB):
                n = k * _SB + i

                for r in range(_CPW):
                    rv = jnp.full((16,), r, jnp.int32)
                    sv = plsc.load_gather(
                        sb, [jnp.full((16,), n, jnp.int32),
                             jnp.full((16,), c0 + r, jnp.int32)])
                    for v in range(hw // 16):
                        yb[b, i, r, pl.ds(v * 16, 16)] = (
                            xb[b, i, r, pl.ds(v * 16, 16)] * sv)
                    xv = plsc.load_gather(xb.at[b, i], [rv, lanes[nvr - 1]])
                    plsc.store_scatter(yb.at[b, i], [rv, lanes[nvr - 1]],
                                       xv * sv)

            for j in range(2):
                out_copy(k, b, j).start()

            @pl.when(k + 2 < K)
            def _():
                start_in(k + 2, b)

    for b in range(2):
        for j in range(2):
            out_copy(K - 2 + b, b, j).wait()


def kernel(x, s_ca, shuffle_num):
    N, c, h, w = x.shape
    hw = h * w
    G = N // _WAY

    x3 = x.reshape(N, c, hw)
    s2 = s_ca.reshape(N, c)

    sc_kernel = pl.kernel(
        _sc_body,
        out_type=jax.ShapeDtypeStruct((_WAY, 2, G, c, hw), x.dtype),
        mesh=plsc.VectorSubcoreMesh(core_axis_name="c", subcore_axis_name="s"),
        scratch_types=[
            pltpu.VMEM((2, _SB, _CPW, hw), x.dtype),
            pltpu.VMEM((N, c), x.dtype),
            pltpu.VMEM((2, _SB, _CPW, hw), x.dtype),
            pltpu.SemaphoreType.DMA,
            pltpu.SemaphoreType.DMA((2,)),
            pltpu.SemaphoreType.DMA((2,)),
        ],
        compiler_params=pltpu.CompilerParams(use_tc_tiling_on_sc=True,
                                             needs_layout_passes=False),
    )
    out = sc_kernel(x3, s2)
    return out.reshape(2 * N, c, h, w)


# speedup vs baseline: 1.0757x; 1.0757x over previous
import jax
import jax.numpy as jnp
from jax import lax
from jax.experimental import pallas as pl
import jax.experimental.pallas.tpu as pltpu
from jax.experimental.pallas import tpu_sc as plsc

_WAY = 5
_CPW = 24
_SB = 2


def _sc_body(x_hbm, s_hbm, o_hbm, xb, sb, yb, s_sem, in_sem, out_sem):
    N, c, hw = x_hbm.shape
    G = N // _WAY
    wid = lax.axis_index("s") * 2 + lax.axis_index("c")
    c0 = wid * _CPW
    nvr = (hw + 15) // 16

    def start_in(k, b):
        pltpu.make_async_copy(x_hbm.at[pl.ds(k * _SB, _SB), pl.ds(c0, _CPW)],
                              xb.at[b], in_sem.at[b]).start()

    def wait_in(k, b):
        pltpu.make_async_copy(x_hbm.at[pl.ds(k * _SB, _SB), pl.ds(c0, _CPW)],
                              xb.at[b], in_sem.at[b]).wait()

    def out_copy(k, b, j):
        n0 = k * _SB
        way = n0 // G
        g0 = lax.rem(n0, G)
        return pltpu.make_async_copy(
            yb.at[b],
            o_hbm.at[way, j, pl.ds(g0, _SB), pl.ds(c0, _CPW)],
            out_sem.at[b])

    pltpu.make_async_copy(s_hbm, sb, s_sem).start()
    start_in(0, 0)
    start_in(1, 1)
    pltpu.make_async_copy(s_hbm, sb, s_sem).wait()

    lanes = [jnp.minimum(lax.iota(jnp.int32, 16) + v * 16,
                         hw - 16 + lax.iota(jnp.int32, 16))
             for v in range(nvr)]

    K = N // _SB
    @pl.loop(0, K // 2)
    def _(g2):
        for b in range(2):
            k = g2 * 2 + b
            wait_in(k, b)

            @pl.when(k >= 2)
            def _():
                for j in range(2):
                    out_copy(k - 2, b, j).wait()

            for i in range(_SB):
                n = k * _SB + i

                @pl.loop(0, _CPW, unroll=4)
                def _(r):
                    rv = jnp.full((16,), r, jnp.int32)
                    sv = plsc.load_gather(
                        sb, [jnp.full((16,), n, jnp.int32),
                             jnp.full((16,), c0 + r, jnp.int32)])
                    for v in range(hw // 16):
                        yb[b, i, r, pl.ds(v * 16, 16)] = (
                            xb[b, i, r, pl.ds(v * 16, 16)] * sv)
                    xv = plsc.load_gather(xb.at[b, i], [rv, lanes[nvr - 1]])
                    plsc.store_scatter(yb.at[b, i], [rv, lanes[nvr - 1]],
                                       xv * sv)

            for j in range(2):
                out_copy(k, b, j).start()

            @pl.when(k + 2 < K)
            def _():
                start_in(k + 2, b)

    for b in range(2):
        for j in range(2):
            out_copy(K - 2 + b, b, j).wait()


def kernel(x, s_ca, shuffle_num):
    N, c, h, w = x.shape
    hw = h * w
    G = N // _WAY

    x3 = x.reshape(N, c, hw)
    s2 = s_ca.reshape(N, c)

    sc_kernel = pl.kernel(
        _sc_body,
        out_type=jax.ShapeDtypeStruct((_WAY, 2, G, c, hw), x.dtype),
        mesh=plsc.VectorSubcoreMesh(core_axis_name="c", subcore_axis_name="s"),
        scratch_types=[
            pltpu.VMEM((2, _SB, _CPW, hw), x.dtype),
            pltpu.VMEM((N, c), x.dtype),
            pltpu.VMEM((2, _SB, _CPW, hw), x.dtype),
            pltpu.SemaphoreType.DMA,
            pltpu.SemaphoreType.DMA((2,)),
            pltpu.SemaphoreType.DMA((2,)),
        ],
        compiler_params=pltpu.CompilerParams(use_tc_tiling_on_sc=True,
                                             needs_layout_passes=False),
    )
    out = sc_kernel(x3, s2)
    return out.reshape(2 * N, c, h, w)


# speedup vs baseline: 1.0831x; 1.0068x over previous
import jax
import jax.numpy as jnp
from jax import lax
from jax.experimental import pallas as pl
import jax.experimental.pallas.tpu as pltpu
from jax.experimental.pallas import tpu_sc as plsc

_WAY = 5
_CPW = 24
_SB = 2


def _sc_body(x_hbm, s_hbm, o_hbm, xb, sb, yb, s_sem, in_sem, out_sem):
    N, c, hw = x_hbm.shape
    G = N // _WAY
    wid = lax.axis_index("s") * 2 + lax.axis_index("c")
    c0 = wid * _CPW
    nvr = (hw + 15) // 16

    def start_in(k, b):
        pltpu.make_async_copy(x_hbm.at[pl.ds(k * _SB, _SB), pl.ds(c0, _CPW)],
                              xb.at[b], in_sem.at[b]).start()

    def wait_in(k, b):
        pltpu.make_async_copy(x_hbm.at[pl.ds(k * _SB, _SB), pl.ds(c0, _CPW)],
                              xb.at[b], in_sem.at[b]).wait()

    def out_copy(k, b, j):
        n0 = k * _SB
        way = n0 // G
        g0 = lax.rem(n0, G)
        return pltpu.make_async_copy(
            yb.at[b],
            o_hbm.at[way, j, pl.ds(g0, _SB), pl.ds(c0, _CPW)],
            out_sem.at[b])

    pltpu.make_async_copy(s_hbm, sb, s_sem).start()
    start_in(0, 0)
    start_in(1, 1)
    pltpu.make_async_copy(s_hbm, sb, s_sem).wait()

    lanes = [jnp.minimum(lax.iota(jnp.int32, 16) + v * 16,
                         hw - 16 + lax.iota(jnp.int32, 16))
             for v in range(nvr)]

    K = N // _SB
    @pl.loop(0, K // 2)
    def _(g2):
        for b in range(2):
            k = g2 * 2 + b
            wait_in(k, b)

            @pl.when(k >= 2)
            def _():
                for j in range(2):
                    out_copy(k - 2, b, j).wait()

            for i in range(_SB):
                n = k * _SB + i

                @pl.loop(0, _CPW)
                def _(r):
                    rv = jnp.full((16,), r, jnp.int32)
                    sv = plsc.load_gather(
                        sb, [jnp.full((16,), n, jnp.int32),
                             jnp.full((16,), c0 + r, jnp.int32)])
                    for v in range(hw // 16):
                        yb[b, i, r, pl.ds(v * 16, 16)] = (
                            xb[b, i, r, pl.ds(v * 16, 16)] * sv)
                    xv = plsc.load_gather(xb.at[b, i], [rv, lanes[nvr - 1]])
                    plsc.store_scatter(yb.at[b, i], [rv, lanes[nvr - 1]],
                                       xv * sv)

            for j in range(2):
                out_copy(k, b, j).start()

            @pl.when(k + 2 < K)
            def _():
                start_in(k + 2, b)

    for b in range(2):
        for j in range(2):
            out_copy(K - 2 + b, b, j).wait()


def kernel(x, s_ca, shuffle_num):
    N, c, h, w = x.shape
    hw = h * w
    G = N // _WAY

    x3 = x.reshape(N, c, hw)
    s2 = s_ca.reshape(N, c)

    sc_kernel = pl.kernel(
        _sc_body,
        out_type=jax.ShapeDtypeStruct((_WAY, 2, G, c, hw), x.dtype),
        mesh=plsc.VectorSubcoreMesh(core_axis_name="c", subcore_axis_name="s"),
        scratch_types=[
            pltpu.VMEM((2, _SB, _CPW, hw), x.dtype),
            pltpu.VMEM((N, c), x.dtype),
            pltpu.VMEM((2, _SB, _CPW, hw), x.dtype),
            pltpu.SemaphoreType.DMA,
            pltpu.SemaphoreType.DMA((2,)),
            pltpu.SemaphoreType.DMA((2,)),
        ],
        compiler_params=pltpu.CompilerParams(use_tc_tiling_on_sc=True,
                                             needs_layout_passes=False),
    )
    out = sc_kernel(x3, s2)
    return out.reshape(2 * N, c, h, w)
